# Initial kernel scaffold; baseline (speedup 1.0000x reference)
import jax
import jax.numpy as jnp
from jax.experimental import pallas as pl


def kernel(x, batch, q_star, bank_s_list, bank_s, index, cuda, W_ih, W_hh, b_ih, b_hh):
    raise NotImplementedError("write your pallas kernel here")


# speedup vs baseline: 6.0599x; 6.0599x over previous
import functools

import jax
import jax.numpy as jnp
from jax.experimental import pallas as pl
from jax.experimental.pallas import tpu as pltpu

_C = 512
_B = 8
_L = 1024
_NTOK = _B * _L
_STEPS = 3
_CHUNK = 512
_NCHUNK = _NTOK // _CHUNK
_PREC = jax.lax.Precision.HIGHEST
_PREC_MM = jax.lax.Precision.DEFAULT


def _fused_body(x_ref, batch_ref, qstar_ref, w_ref, wih_ref, whh_ref, b_ref,
                out_ref, st_ref, h_ref):
    seg_row = batch_ref[...]

    oh_t = (jax.lax.broadcasted_iota(jnp.int32, (_B, _NTOK), 0)
            == seg_row).astype(jnp.float32)

    def asit_step(i, _):
        wrow = w_ref[pl.ds(i, 1), :]
        segx = x_ref[pl.ds(i * _L, _L), :]
        h_ref[pl.ds(i, 1), :] = jax.lax.dot_general(
            wrow, segx, (((1,), (0,)), ((), ())), precision=_PREC_MM)
        return 0

    jax.lax.fori_loop(0, _B, asit_step, 0)
    h = h_ref[...]
    c = jnp.zeros((_B, _C), jnp.float32)
    qs = qstar_ref[...]
    bias = b_ref[...]

    for _ in range(_STEPS):
        def gate_chunk(g, _):
            wih_c = wih_ref[pl.ds(g * _CHUNK, _CHUNK), :]
            whh_c = whh_ref[pl.ds(g * _CHUNK, _CHUNK), :]
            st_ref[:, pl.ds(g * _CHUNK, _CHUNK)] = (
                jax.lax.dot_general(qs, wih_c, (((1,), (1,)), ((), ())),
                                    precision=_PREC_MM)
                + jax.lax.dot_general(h, whh_c, (((1,), (1,)), ((), ())),
                                      precision=_PREC_MM))
            return 0

        jax.lax.fori_loop(0, (4 * _C) // _CHUNK, gate_chunk, 0)
        gates = st_ref[:, pl.ds(0, 4 * _C)] + bias
        ig = jax.nn.sigmoid(gates[:, 0 * _C:1 * _C])
        fg = jax.nn.sigmoid(gates[:, 1 * _C:2 * _C])
        gg = jnp.tanh(gates[:, 2 * _C:3 * _C])
        og = jax.nn.sigmoid(gates[:, 3 * _C:4 * _C])
        c = fg * c + ig * gg
        h = og * jnp.tanh(c)

        def score_chunk(j, _):
            xc = x_ref[pl.ds(j * _CHUNK, _CHUNK), :]
            st_ref[:, pl.ds(j * _CHUNK, _CHUNK)] = jax.lax.dot_general(
                h, xc, (((1,), (1,)), ((), ())), precision=_PREC)
            return 0

        jax.lax.fori_loop(0, _NCHUNK, score_chunk, 0)
        s_t = st_ref[...]

        smask = jnp.where(oh_t > 0.0, s_t, -jnp.inf)
        seg_max = jnp.max(smask, axis=1, keepdims=True)
        z_row = jnp.sum(oh_t * (s_t - seg_max), axis=0, keepdims=True)
        e_row = jnp.exp(z_row)
        denom = jnp.sum(oh_t * e_row, axis=1, keepdims=True)
        denom_row = jnp.sum(oh_t * denom, axis=0, keepdims=True)
        a_row = e_row / (denom_row + 1e-16)
        st_ref[...] = oh_t * a_row

        def pool_chunk(j, acc):
            wac = st_ref[:, pl.ds(j * _CHUNK, _CHUNK)]
            xc = x_ref[pl.ds(j * _CHUNK, _CHUNK), :]
            return acc + jax.lax.dot_general(
                wac, xc, (((1,), (0,)), ((), ())), precision=_PREC)

        r = jax.lax.fori_loop(0, _NCHUNK, pool_chunk,
                              jnp.zeros((_B, _C), jnp.float32))
        qs = jnp.concatenate([h, r], axis=1)

    out_ref[...] = qs


@functools.partial(jax.jit, static_argnames=("interpret",))
def _run_fused(x, seg_row, q_star, w_rows, W_ih, W_hh, bias, interpret=False):
    return pl.pallas_call(
        _fused_body,
        out_shape=jax.ShapeDtypeStruct((_B, 2 * _C), jnp.float32),
        scratch_shapes=[
            pltpu.VMEM((_B, _NTOK), jnp.float32),
            pltpu.VMEM((_B, _C), jnp.float32),
        ],
        interpret=interpret,
    )(x, seg_row, q_star, w_rows, W_ih, W_hh, bias)


def kernel(x, batch, q_star, bank_s_list, bank_s, index, cuda,
           W_ih, W_hh, b_ih, b_hh, interpret=False):
    w_rows = jax.lax.dynamic_slice_in_dim(
        bank_s_list, index, 1, axis=1).reshape(_B, _L)
    seg_row = batch.astype(jnp.int32).reshape(1, _NTOK)
    bias = jnp.broadcast_to((b_ih + b_hh).reshape(1, 4 * _C), (_B, 4 * _C))
    return _run_fused(x, seg_row, q_star, w_rows, W_ih, W_hh, bias,
                      interpret=interpret)
